# baseline (speedup 1.0000x reference)
import functools

import jax
import jax.numpy as jnp
from jax import lax
from jax.experimental import pallas as pl
from jax.experimental.pallas import tpu as pltpu
from jax.experimental.pallas import tpu_sc as plsc

_USERS = 4000
_ITEMS = 6000
_V = 10000
_D = 128
_D2 = 64
_E = 320000
_LAYERS = 3
_NS = 16
_CHUNK = 125
_GSZ = 32
_NG = 5
_EPT = _NG * _GSZ
_TPR = 640
_VB = 10048
_FCH = 40

_mesh = plsc.VectorSubcoreMesh(core_axis_name="c", subcore_axis_name="s")


@functools.partial(
    pl.kernel,
    out_type=(jax.ShapeDtypeStruct((_USERS, _D), jnp.float32),
              jax.ShapeDtypeStruct((_ITEMS, _D), jnp.float32)),
    mesh=_mesh,
    compiler_params=pltpu.CompilerParams(use_tc_tiling_on_sc=False),
    scratch_types=[
        pltpu.VMEM_SHARED((_VB, _D2), jnp.float32),
        pltpu.VMEM_SHARED((_VB, _D2), jnp.float32),
        pltpu.VMEM((_GSZ, _CHUNK), jnp.int32),
        pltpu.VMEM((_GSZ, _CHUNK), jnp.int32),
        [pltpu.VMEM((_CHUNK, _D2), jnp.float32) for _ in range(4)],
        pltpu.VMEM((_FCH, _D2), jnp.float32),
        pltpu.VMEM((_FCH, _D2), jnp.float32),
        [pltpu.SemaphoreType.DMA for _ in range(4)],
        [pltpu.SemaphoreType.DMA for _ in range(4)],
    ],
)
def _lightgcn(src_hbm, dst_hbm, emb_hbm, zeros_hbm, user_hbm, item_hbm,
              xa, xb, sidx, didx, bufs, obuf, pbuf, gsems, ssems):
    t = lax.axis_index("s")
    cid = lax.axis_index("c")
    base = pl.multiple_of(t * _TPR, _TPR)
    nch = (jnp.minimum(base + _TPR, _V) - base) // _FCH
    zbase = pl.multiple_of(jnp.minimum(base, _VB - _TPR), 8)
    csl = pl.ds(cid * _D2, _D2)

    pltpu.sync_copy(emb_hbm.at[pl.ds(zbase, _TPR), csl],
                    xa.at[pl.ds(zbase, _TPR)])
    pltpu.sync_copy(zeros_hbm, xb.at[pl.ds(zbase, _TPR)])

    plsc.subcore_barrier()

    for layer in range(_LAYERS):
        last = layer == _LAYERS - 1
        xsrc, xdst = (xa, xb) if layer % 2 == 0 else (xb, xa)

        @pl.loop(0, _NG)
        def _edge_group(g):
            gsl = pl.ds(pl.multiple_of(g * _GSZ, _GSZ), _GSZ)
            pltpu.sync_copy(src_hbm.at[t, gsl], sidx)
            pltpu.sync_copy(dst_hbm.at[t, gsl], didx)
            gd, sd = {}, {}
            for k in range(2):
                gd[k] = pltpu.async_copy(
                    xsrc.at[sidx.at[k]], bufs[k], gsems[k])
            for j in range(_GSZ):
                if j >= 2:
                    sd[j - 2].wait()
                if j + 2 < _GSZ:
                    b = (j + 2) % 4
                    gd[j + 2] = pltpu.async_copy(
                        xsrc.at[sidx.at[j + 2]], bufs[b], gsems[b])
                gd[j].wait()
                sd[j] = pltpu.async_copy(
                    bufs[j % 4], xdst.at[didx.at[j]], ssems[j % 4],
                    add=True)
            sd[_GSZ - 2].wait()
            sd[_GSZ - 1].wait()

        plsc.subcore_barrier()

        if not last:
            pltpu.sync_copy(zeros_hbm, xsrc.at[pl.ds(zbase, _TPR)])

        @pl.loop(0, nch)
        def _flush(c):
            r0 = pl.multiple_of(base + c * _FCH, _FCH)
            psl = pl.ds(r0, _FCH)
            pltpu.sync_copy(xdst.at[psl], pbuf)

            def _out_slice(ref, roff):
                return ref.at[pl.ds(pl.multiple_of(r0 - roff, 8), _FCH), csl]

            def _with_out(fn):
                @pl.when(r0 < _USERS)
                def _u():
                    fn(_out_slice(user_hbm, 0))

                @pl.when(r0 >= _USERS)
                def _i():
                    fn(_out_slice(item_hbm, _USERS))

            if layer == 0:
                pltpu.sync_copy(emb_hbm.at[psl, csl], obuf)
            else:
                _with_out(lambda o: pltpu.sync_copy(o, obuf))

            @pl.loop(0, _FCH)
            def _acc_row(r):
                for cc in range(_D2 // 16):
                    sl = pl.ds(cc * 16, 16)
                    s = obuf[r, sl] + pbuf[r, sl]
                    if last:
                        s = s * 0.25
                    obuf[r, sl] = s

            _with_out(lambda o: pltpu.sync_copy(obuf, o))

        if not last:
            plsc.subcore_barrier()


def kernel(edge_index, emb_weight):
    src_p = edge_index[0].reshape(_NS, _EPT, _CHUNK)
    dst_p = edge_index[1].reshape(_NS, _EPT, _CHUNK)
    zeros = jnp.zeros((_TPR, _D2), jnp.float32)
    return _lightgcn(src_p, dst_p, emb_weight, zeros)


# speedup vs baseline: 1.0767x; 1.0767x over previous
import functools

import jax
import jax.numpy as jnp
from jax import lax
from jax.experimental import pallas as pl
from jax.experimental.pallas import tpu as pltpu
from jax.experimental.pallas import tpu_sc as plsc

_USERS = 4000
_V = 10000
_D = 128
_D2 = 64
_E = 320000
_LAYERS = 3
_NS = 16
_CHUNK = 125
_GSZ = 32
_NG = 5
_EPT = _NG * _GSZ
_TPR = 640
_VB = 10048
_FCH = 40

_mesh = plsc.VectorSubcoreMesh(core_axis_name="c", subcore_axis_name="s")


@functools.partial(
    pl.kernel,
    out_type=jax.ShapeDtypeStruct((_V, _D), jnp.float32),
    mesh=_mesh,
    compiler_params=pltpu.CompilerParams(use_tc_tiling_on_sc=False),
    scratch_types=[
        pltpu.VMEM_SHARED((_VB, _D2), jnp.float32),
        pltpu.VMEM_SHARED((_VB, _D2), jnp.float32),
        pltpu.VMEM((_GSZ, _CHUNK), jnp.int32),
        pltpu.VMEM((_GSZ, _CHUNK), jnp.int32),
        [pltpu.VMEM((_CHUNK, _D2), jnp.float32) for _ in range(4)],
        [pltpu.VMEM((_FCH, _D2), jnp.float32) for _ in range(2)],
        [pltpu.VMEM((_FCH, _D2), jnp.float32) for _ in range(2)],
        [pltpu.SemaphoreType.DMA for _ in range(4)],
        [pltpu.SemaphoreType.DMA for _ in range(4)],
        [pltpu.SemaphoreType.DMA for _ in range(2)],
    ],
)
def _lightgcn(src_hbm, dst_hbm, emb_hbm, zeros_hbm, out_hbm,
              xa, xb, sidx, didx, bufs, obuf, pbuf, gsems, ssems, isems):
    t = lax.axis_index("s")
    cid = lax.axis_index("c")
    base = pl.multiple_of(t * _TPR, _TPR)
    nch2 = (jnp.minimum(base + _TPR, _V) - base) // (2 * _FCH)
    zbase = pl.multiple_of(jnp.minimum(base, _V - _TPR), 8)
    csl = pl.ds(cid * _D2, _D2)
    zsl = pl.ds(zbase, _TPR)

    ph = pltpu.async_copy(emb_hbm.at[zsl, csl], xa.at[zsl], gsems[0])
    zh = pltpu.async_copy(zeros_hbm, xb.at[zsl], gsems[1])
    ph.wait()
    zh.wait()

    plsc.subcore_barrier()

    for layer in range(_LAYERS):
        last = layer == _LAYERS - 1
        xsrc, xdst = (xa, xb) if layer % 2 == 0 else (xb, xa)

        @pl.loop(0, _NG)
        def _edge_group(g):
            gsl = pl.ds(pl.multiple_of(g * _GSZ, _GSZ), _GSZ)
            ih0 = pltpu.async_copy(src_hbm.at[t, gsl], sidx, isems[0])
            ih1 = pltpu.async_copy(dst_hbm.at[t, gsl], didx, isems[1])
            ih0.wait()
            gd, sd = {}, {}
            for k in range(2):
                gd[k] = pltpu.async_copy(
                    xsrc.at[sidx.at[k]], bufs[k], gsems[k])
            ih1.wait()
            for j in range(_GSZ):
                if j >= 2:
                    sd[j - 2].wait()
                if j + 2 < _GSZ:
                    b = (j + 2) % 4
                    gd[j + 2] = pltpu.async_copy(
                        xsrc.at[sidx.at[j + 2]], bufs[b], gsems[b])
                gd[j].wait()
                sd[j] = pltpu.async_copy(
                    bufs[j % 4], xdst.at[didx.at[j]], ssems[j % 4],
                    add=True)
            sd[_GSZ - 2].wait()
            sd[_GSZ - 1].wait()

        plsc.subcore_barrier()

        rz = None
        if not last:
            rz = pltpu.async_copy(zeros_hbm, xsrc.at[zsl], isems[0])

        @pl.loop(0, nch2)
        def _flush(c):
            orh, prh, owh = {}, {}, {}
            for h in range(2):
                r0 = pl.multiple_of(base + (2 * c + h) * _FCH, _FCH)
                prh[h] = pltpu.async_copy(
                    xdst.at[pl.ds(r0, _FCH)], pbuf[h], gsems[2 + h])
                src_ref = emb_hbm if layer == 0 else out_hbm
                orh[h] = pltpu.async_copy(
                    src_ref.at[pl.ds(r0, _FCH), csl], obuf[h], gsems[h])
            for h in range(2):
                r0 = pl.multiple_of(base + (2 * c + h) * _FCH, _FCH)
                orh[h].wait()
                prh[h].wait()

                @pl.loop(0, _FCH)
                def _acc_row(r):
                    for cc in range(_D2 // 16):
                        sl = pl.ds(cc * 16, 16)
                        s = obuf[h][r, sl] + pbuf[h][r, sl]
                        if last:
                            s = s * 0.25
                        obuf[h][r, sl] = s

                owh[h] = pltpu.async_copy(
                    obuf[h], out_hbm.at[pl.ds(r0, _FCH), csl], ssems[h])
            owh[0].wait()
            owh[1].wait()

        if rz is not None:
            rz.wait()
            plsc.subcore_barrier()


def kernel(edge_index, emb_weight):
    src_p = edge_index[0].reshape(_NS, _EPT, _CHUNK)
    dst_p = edge_index[1].reshape(_NS, _EPT, _CHUNK)
    zeros = jnp.zeros((_TPR, _D2), jnp.float32)
    out = _lightgcn(src_p, dst_p, emb_weight, zeros)
    return out[:_USERS], out[_USERS:]
